# Initial kernel scaffold; baseline (speedup 1.0000x reference)
import jax
import jax.numpy as jnp
from jax.experimental import pallas as pl


def kernel(x, edge_index, W_l1, b_l1, W_r1, g1, be1, W_l2, b_l2, W_r2, g2, be2, Wn1, bn1, Wn2, bn2):
    raise NotImplementedError("write your pallas kernel here")


# speedup vs baseline: 5.4048x; 5.4048x over previous
import functools

import jax
import jax.numpy as jnp
from jax import lax
from jax.experimental import pallas as pl
from jax.experimental.pallas import tpu as pltpu
from jax.experimental.pallas import tpu_sc as plsc

_N = 10000
_E = 320000
_D_IN = 128
_HID = 64
_OUT = 32

_NC = 2
_NS = 16
_NW = _NC * _NS
_CH = 128
_CPAD = 2560
_NPAD = 10240
_RPT = _NPAD // _NS
_JPW = _CPAD // _NW
_DUMP = _N

_f32 = jnp.float32


def _sc_edge_body(nfeat, with_counts, *refs):
    if with_counts:
        (y_hbm, src_hbm, dst_hbm, zrow_hbm, z16_hbm, o16_hbm,
         sums_hbm, cnts_hbm, acc, cnt, sidx, didx, rows, onesv, sem) = refs
    else:
        (y_hbm, src_hbm, dst_hbm, zrow_hbm,
         sums_hbm, acc, sidx, didx, rows, sem) = refs

    c = lax.axis_index("c")
    s = lax.axis_index("s")
    wid = s * _NC + c

    pltpu.sync_copy(zrow_hbm, rows)
    if with_counts:
        pltpu.sync_copy(z16_hbm, onesv)
    for k in range(_RPT // _CH):
        r0 = s * _RPT + k * _CH
        pltpu.sync_copy(rows, acc.at[pl.ds(r0, _CH)])
        if with_counts:
            pltpu.sync_copy(onesv, cnt.at[pl.ds(r0, _CH)])
    if with_counts:
        pltpu.sync_copy(o16_hbm, onesv)

    start = _JPW * wid
    pltpu.sync_copy(src_hbm.at[pl.ds(start, _JPW)], sidx)
    pltpu.sync_copy(dst_hbm.at[pl.ds(start, _JPW)], didx)

    plsc.subcore_barrier()

    def chunk(t, carry):
        pltpu.async_copy(y_hbm.at[sidx.at[t]], rows, sem).wait()
        pltpu.sync_copy(rows, acc.at[didx.at[t]], add=True)
        if with_counts:
            pltpu.sync_copy(onesv, cnt.at[didx.at[t]], add=True)
        return carry

    lax.fori_loop(0, _JPW, chunk, 0)

    plsc.subcore_barrier()

    for k in range(_RPT // _CH):
        r0 = s * _RPT + k * _CH
        pltpu.sync_copy(acc.at[pl.ds(r0, _CH)], rows)
        pltpu.sync_copy(rows, sums_hbm.at[pl.ds(c * _NPAD + r0, _CH)])
        if with_counts:
            pltpu.sync_copy(cnt.at[pl.ds(r0, _CH)], onesv)
            pltpu.sync_copy(onesv, cnts_hbm.at[pl.ds(c * _NPAD + r0, _CH)])


def _make_sc_layer1():
    mesh = plsc.VectorSubcoreMesh(core_axis_name="c", subcore_axis_name="s")
    return pl.kernel(
        functools.partial(_sc_edge_body, _HID, True),
        compiler_params=pltpu.CompilerParams(use_tc_tiling_on_sc=False),
        out_type=(
            jax.ShapeDtypeStruct((_NC * _NPAD, _HID), _f32),
            jax.ShapeDtypeStruct((_NC * _NPAD, 16), _f32),
        ),
        mesh=mesh,
        scratch_types=[
            pltpu.VMEM_SHARED((_NPAD, _HID), _f32),
            pltpu.VMEM_SHARED((_NPAD, 16), _f32),
            pltpu.VMEM((_JPW, _CH), jnp.int32),
            pltpu.VMEM((_JPW, _CH), jnp.int32),
            pltpu.VMEM((_CH, _HID), _f32),
            pltpu.VMEM((_CH, 16), _f32),
            pltpu.SemaphoreType.DMA,
        ],
    )


def _make_sc_layer2():
    mesh = plsc.VectorSubcoreMesh(core_axis_name="c", subcore_axis_name="s")
    return pl.kernel(
        functools.partial(_sc_edge_body, _OUT, False),
        compiler_params=pltpu.CompilerParams(use_tc_tiling_on_sc=False),
        out_type=jax.ShapeDtypeStruct((_NC * _NPAD, _OUT), _f32),
        mesh=mesh,
        scratch_types=[
            pltpu.VMEM_SHARED((_NPAD, _OUT), _f32),
            pltpu.VMEM((_JPW, _CH), jnp.int32),
            pltpu.VMEM((_JPW, _CH), jnp.int32),
            pltpu.VMEM((_CH, _OUT), _f32),
            pltpu.SemaphoreType.DMA,
        ],
    )


def _tc_a_body(x_ref, w_ref, y_ref, r_ref):
    yr = jnp.dot(x_ref[...], w_ref[...].T, preferred_element_type=_f32)
    y_ref[...] = yr[:, :_HID]
    r_ref[...] = yr[:, _HID:]


def _tc_b_body(sums_ref, cnts_ref, r1_ref, bl1_ref, g1_ref, be1_ref, w2_ref,
               y2_ref, r2_ref):
    s = sums_ref[0:_N, :] + sums_ref[_NPAD:_NPAD + _N, :]
    cn = cnts_ref[0:_N, :] + cnts_ref[_NPAD:_NPAD + _N, :]
    c = jnp.maximum(cn[:, 0:1], 1.0)
    pre = s / c + bl1_ref[...] + r1_ref[...]
    mu = jnp.mean(pre, axis=0, keepdims=True)
    var = jnp.mean((pre - mu) * (pre - mu), axis=0, keepdims=True)
    h1 = jnp.maximum((pre - mu) * lax.rsqrt(var + 1e-5) * g1_ref[...]
                     + be1_ref[...], 0.0)
    yr2 = jnp.dot(h1, w2_ref[...].T, preferred_element_type=_f32)
    y2_ref[...] = yr2[:, :_OUT]
    r2_ref[...] = yr2[:, _OUT:]


def _tc_c_body(sums_ref, cnts_ref, r2_ref, bl2_ref, g2_ref, be2_ref,
               wn1_ref, bn1_ref, wn2_ref, bn2_ref, h_ref, risk_ref):
    s = sums_ref[0:_N, :] + sums_ref[_NPAD:_NPAD + _N, :]
    cn = cnts_ref[0:_N, :] + cnts_ref[_NPAD:_NPAD + _N, :]
    c = jnp.maximum(cn[:, 0:1], 1.0)
    pre = s / c + bl2_ref[...] + r2_ref[...]
    mu = jnp.mean(pre, axis=0, keepdims=True)
    var = jnp.mean((pre - mu) * (pre - mu), axis=0, keepdims=True)
    h2 = jnp.maximum((pre - mu) * lax.rsqrt(var + 1e-5) * g2_ref[...]
                     + be2_ref[...], 0.0)
    h_ref[...] = h2
    z = jnp.maximum(jnp.dot(h2, wn1_ref[...].T, preferred_element_type=_f32)
                    + bn1_ref[...], 0.0)
    logit = jnp.sum(z * wn2_ref[...], axis=1, keepdims=True) + bn2_ref[...]
    risk_ref[...] = jax.nn.sigmoid(logit)


def _tc_a(x, w1cat):
    return pl.pallas_call(
        _tc_a_body,
        out_shape=(
            jax.ShapeDtypeStruct((_N, _HID), _f32),
            jax.ShapeDtypeStruct((_N, _HID), _f32),
        ),
    )(x, w1cat)


def _tc_b(sums, cnts, r1, bl1, g1, be1, w2cat):
    return pl.pallas_call(
        _tc_b_body,
        out_shape=(
            jax.ShapeDtypeStruct((_N, _OUT), _f32),
            jax.ShapeDtypeStruct((_N, _OUT), _f32),
        ),
    )(sums, cnts, r1, bl1, g1, be1, w2cat)


def _tc_c(sums, cnts, r2, bl2, g2, be2, wn1, bn1, wn2, bn2):
    return pl.pallas_call(
        _tc_c_body,
        out_shape=(
            jax.ShapeDtypeStruct((_N, _OUT), _f32),
            jax.ShapeDtypeStruct((_N, 1), _f32),
        ),
    )(sums, cnts, r2, bl2, g2, be2, wn1, bn1, wn2, bn2)


@jax.jit
def _run(x, edge_index, W_l1, b_l1, W_r1, g1, be1, W_l2, b_l2, W_r2, g2, be2,
         Wn1, bn1, Wn2, bn2):
    pad = _CPAD * _CH - _E
    src2d = jnp.concatenate(
        [edge_index[0], jnp.zeros((pad,), jnp.int32)]).reshape(_CPAD, _CH)
    dst2d = jnp.concatenate(
        [edge_index[1], jnp.full((pad,), _DUMP, jnp.int32)]).reshape(_CPAD, _CH)

    zrow1 = jnp.zeros((_CH, _HID), _f32)
    z16 = jnp.zeros((_CH, 16), _f32)
    o16 = jnp.ones((_CH, 16), _f32)
    zrow2 = jnp.zeros((_CH, _OUT), _f32)

    w1cat = jnp.concatenate([W_l1, W_r1], axis=0)
    w2cat = jnp.concatenate([W_l2, W_r2], axis=0)

    y1, r1 = _tc_a(x, w1cat)

    sums1, cnts = _make_sc_layer1()(y1, src2d, dst2d, zrow1, z16, o16)

    y2, r2 = _tc_b(sums1, cnts, r1,
                   b_l1.reshape(1, _HID), g1.reshape(1, _HID),
                   be1.reshape(1, _HID), w2cat)

    sums2 = _make_sc_layer2()(y2, src2d, dst2d, zrow2)

    h, risk2d = _tc_c(sums2, cnts, r2,
                      b_l2.reshape(1, _OUT), g2.reshape(1, _OUT),
                      be2.reshape(1, _OUT),
                      Wn1, bn1.reshape(1, 16), Wn2, bn2.reshape(1, 1))
    return h, risk2d[:, 0]


def kernel(x, edge_index, W_l1, b_l1, W_r1, g1, be1, W_l2, b_l2, W_r2, g2,
           be2, Wn1, bn1, Wn2, bn2):
    return _run(x, edge_index, W_l1, b_l1, W_r1, g1, be1, W_l2, b_l2, W_r2,
                g2, be2, Wn1, bn1, Wn2, bn2)
